# Initial kernel scaffold; baseline (speedup 1.0000x reference)
import functools

import jax
import jax.numpy as jnp
from jax import lax
from jax.experimental import pallas as pl
from jax.experimental.pallas import tpu as pltpu
from jax.experimental.pallas import tpu_sc as plsc


def _sc_gather(table, idx, num_cores, num_subcores, chunk):
    n = idx.shape[0]
    d = table.shape[1]
    nw = num_cores * num_subcores
    per_w = n // nw
    steps = per_w // chunk
    mesh = plsc.VectorSubcoreMesh(core_axis_name="c", subcore_axis_name="s")

    @functools.partial(
        pl.kernel,
        mesh=mesh,
        out_type=jax.ShapeDtypeStruct((n, d), jnp.float32),
        scratch_types=[
            pltpu.VMEM((chunk,), jnp.int32),
            pltpu.VMEM((chunk, d), jnp.float32),
            pltpu.SemaphoreType.DMA,
        ],
    )
    def k(idx_hbm, table_hbm, out_hbm, idx_v, rows_v, sem):
        wid = lax.axis_index("s") * num_cores + lax.axis_index("c")
        base = wid * per_w

        @pl.loop(0, steps)
        def _(g):
            off = base + g * chunk
            pltpu.sync_copy(idx_hbm.at[pl.ds(off, chunk)], idx_v)
            pltpu.async_copy(table_hbm.at[idx_v], rows_v, sem).wait()
            pltpu.sync_copy(rows_v, out_hbm.at[pl.ds(off, chunk)])

    return k(idx, table)


def kernel(tokens, table):
    b, h = tokens.shape
    d = table.shape[1]
    idx = tokens.reshape(b * h).astype(jnp.int32)
    info = plsc.get_sparse_core_info()
    out = _sc_gather(table, idx, info.num_cores, info.num_subcores, 128)
    return out.reshape(b, h, d)


# speedup vs baseline: 1.5712x; 1.5712x over previous
import functools

import jax
import jax.numpy as jnp
from jax import lax
from jax.experimental import pallas as pl
from jax.experimental.pallas import tpu as pltpu
from jax.experimental.pallas import tpu_sc as plsc


def _sc_gather(table, idx, num_cores, num_subcores, chunk):
    n = idx.shape[0]
    d = table.shape[1]
    nw = num_cores * num_subcores
    per_w = n // nw
    steps = per_w // chunk
    mesh = plsc.VectorSubcoreMesh(core_axis_name="c", subcore_axis_name="s")

    @functools.partial(
        pl.kernel,
        mesh=mesh,
        out_type=jax.ShapeDtypeStruct((n, d), jnp.float32),
        scratch_types=[
            pltpu.VMEM((chunk,), jnp.int32),
            pltpu.VMEM((chunk, d), jnp.float32),
            pltpu.SemaphoreType.DMA,
        ],
        compiler_params=pltpu.CompilerParams(use_tc_tiling_on_sc=False),
    )
    def k(idx_hbm, table_hbm, out_hbm, idx_v, rows_v, sem):
        wid = lax.axis_index("s") * num_cores + lax.axis_index("c")
        base = wid * per_w

        @pl.loop(0, steps)
        def _(g):
            off = base + g * chunk
            pltpu.sync_copy(idx_hbm.at[pl.ds(off, chunk)], idx_v)
            pltpu.async_copy(table_hbm.at[idx_v], rows_v, sem).wait()
            pltpu.sync_copy(rows_v, out_hbm.at[pl.ds(off, chunk)])

    return k(idx, table)


def kernel(tokens, table):
    b, h = tokens.shape
    d = table.shape[1]
    idx = tokens.reshape(b * h).astype(jnp.int32)
    info = plsc.get_sparse_core_info()
    out = _sc_gather(table, idx, info.num_cores, info.num_subcores, 128)
    return out.reshape(b, h, d)


# speedup vs baseline: 1.8717x; 1.1913x over previous
import functools

import jax
import jax.numpy as jnp
from jax import lax
from jax.experimental import pallas as pl
from jax.experimental.pallas import tpu as pltpu
from jax.experimental.pallas import tpu_sc as plsc


def _sc_gather(table, idx, num_cores, num_subcores, chunk, nbuf):
    n = idx.shape[0]
    d = table.shape[1]
    nw = num_cores * num_subcores
    per_w = n // nw
    steps = per_w // chunk
    assert steps % nbuf == 0 and steps >= 2 * nbuf
    mesh = plsc.VectorSubcoreMesh(core_axis_name="c", subcore_axis_name="s")

    @functools.partial(
        pl.kernel,
        mesh=mesh,
        out_type=jax.ShapeDtypeStruct((n, d), jnp.float32),
        scratch_types=[
            [pltpu.VMEM((chunk,), jnp.int32) for _ in range(nbuf)],
            [pltpu.VMEM((chunk, d), jnp.float32) for _ in range(nbuf)],
            [pltpu.SemaphoreType.DMA for _ in range(nbuf)],
            [pltpu.SemaphoreType.DMA for _ in range(nbuf)],
            [pltpu.SemaphoreType.DMA for _ in range(nbuf)],
        ],
        compiler_params=pltpu.CompilerParams(use_tc_tiling_on_sc=False),
    )
    def k(idx_hbm, table_hbm, out_hbm, idx_v, rows_v, sem_i, sem_g, sem_s):
        wid = lax.axis_index("s") * num_cores + lax.axis_index("c")
        base = wid * per_w

        def idx_load(b, g):
            return pltpu.make_async_copy(
                idx_hbm.at[pl.ds(base + g * chunk, chunk)], idx_v[b], sem_i[b]
            )

        def gather(b):
            return pltpu.make_async_copy(
                table_hbm.at[idx_v[b]], rows_v[b], sem_g[b]
            )

        def store(b, g):
            return pltpu.make_async_copy(
                rows_v[b], out_hbm.at[pl.ds(base + g * chunk, chunk)], sem_s[b]
            )

        def body(g0, first, last):
            for b in range(nbuf):
                g = g0 + b
                if not first:
                    store(b, g).wait()
                idx_load(b, g).wait()
                gather(b).start()
            for b in range(nbuf):
                g = g0 + b
                gather(b).wait()
                store(b, g).start()
                if not last:
                    idx_load(b, g + nbuf).start()

        for b in range(nbuf):
            idx_load(b, b).start()
        body(0, True, False)

        @pl.loop(nbuf, steps - nbuf, step=nbuf)
        def _(g0):
            body(g0, False, False)

        body(steps - nbuf, False, True)
        for b in range(nbuf):
            store(b, 0).wait()

    return k(idx, table)


def kernel(tokens, table):
    b, h = tokens.shape
    d = table.shape[1]
    idx = tokens.reshape(b * h).astype(jnp.int32)
    info = plsc.get_sparse_core_info()
    out = _sc_gather(table, idx, info.num_cores, info.num_subcores, 128, 4)
    return out.reshape(b, h, d)


# speedup vs baseline: 1.8717x; 1.0000x over previous
import functools

import jax
import jax.numpy as jnp
from jax import lax
from jax.experimental import pallas as pl
from jax.experimental.pallas import tpu as pltpu
from jax.experimental.pallas import tpu_sc as plsc


def _sc_gather(table, idx, num_cores, num_subcores, chunk, nbuf):
    n = idx.shape[0]
    d = table.shape[1]
    nw = num_cores * num_subcores
    per_w = n // nw
    steps = per_w // chunk
    assert steps % nbuf == 0 and steps >= 2 * nbuf
    mesh = plsc.VectorSubcoreMesh(core_axis_name="c", subcore_axis_name="s")

    @functools.partial(
        pl.kernel,
        mesh=mesh,
        out_type=jax.ShapeDtypeStruct((n, d), jnp.float32),
        scratch_types=[
            [pltpu.VMEM((chunk,), jnp.int32) for _ in range(nbuf)],
            [pltpu.VMEM((chunk, d), jnp.float32) for _ in range(nbuf)],
            [pltpu.SemaphoreType.DMA for _ in range(nbuf)],
            [pltpu.SemaphoreType.DMA for _ in range(nbuf)],
            [pltpu.SemaphoreType.DMA for _ in range(nbuf)],
        ],
        compiler_params=pltpu.CompilerParams(use_tc_tiling_on_sc=False),
    )
    def k(idx_hbm, table_hbm, out_hbm, idx_v, rows_v, sem_i, sem_g, sem_s):
        wid = lax.axis_index("s") * num_cores + lax.axis_index("c")
        base = wid * per_w

        def idx_load(b, g):
            return pltpu.make_async_copy(
                idx_hbm.at[pl.ds(base + g * chunk, chunk)], idx_v[b], sem_i[b]
            )

        def gather(b):
            return pltpu.make_async_copy(
                table_hbm.at[idx_v[b]], rows_v[b], sem_g[b]
            )

        def store(b, g):
            return pltpu.make_async_copy(
                rows_v[b], out_hbm.at[pl.ds(base + g * chunk, chunk)], sem_s[b]
            )

        def body(g0, first, last):
            for b in range(nbuf):
                g = g0 + b
                if not first:
                    store(b, g).wait()
                idx_load(b, g).wait()
                gather(b).start()
            for b in range(nbuf):
                g = g0 + b
                gather(b).wait()
                store(b, g).start()
                if not last:
                    idx_load(b, g + nbuf).start()

        for b in range(nbuf):
            idx_load(b, b).start()
        body(0, True, False)

        @pl.loop(nbuf, steps - nbuf, step=nbuf)
        def _(g0):
            body(g0, False, False)

        body(steps - nbuf, False, True)
        for b in range(nbuf):
            store(b, 0).wait()

    return k(idx, table)


def kernel(tokens, table):
    b, h = tokens.shape
    d = table.shape[1]
    idx = tokens.reshape(b * h).astype(jnp.int32)
    info = plsc.get_sparse_core_info()
    out = _sc_gather(table, idx, info.num_cores, info.num_subcores, 256, 4)
    return out.reshape(b, h, d)
